# Initial kernel scaffold; baseline (speedup 1.0000x reference)
import jax
import jax.numpy as jnp
from jax.experimental import pallas as pl


def kernel(x, expert_idx, w1, w3, w2):
    raise NotImplementedError("write your pallas kernel here")


# speedup vs baseline: 2.2294x; 2.2294x over previous
import functools

import jax
import jax.numpy as jnp
from jax import lax
from jax.experimental import pallas as pl
from jax.experimental.pallas import tpu as pltpu

E, D, F = 8, 2048, 5632
T = 256
FB = 256
NF = F // FB
N_TOK = 4096
P = N_TOK + E * T


def _grouped_ffn_body(offs_ref, ntiles_ref, w1_ref, w3_ref, w2_ref,
                      x_hbm, out_hbm, x_vmem, acc_ref, ld_sem, st_sem):
    e = pl.program_id(0)
    f = pl.program_id(1)

    off = offs_ref[e]
    nt = ntiles_ref[e]

    @pl.when(f == 0)
    def _load_seg():
        def stage(k, carry):
            cp = pltpu.make_async_copy(
                x_hbm.at[pl.ds(pl.multiple_of(off + k * T, T), T), :],
                x_vmem.at[pl.ds(pl.multiple_of(k * T, T), T), :],
                ld_sem)
            cp.start()
            cp.wait()
            return carry
        lax.fori_loop(0, nt, stage, 0)

    def tile_body(k, carry):
        rows = x_vmem[pl.ds(pl.multiple_of(k * T, T), T), :]
        g = jnp.dot(rows, w1_ref[0], preferred_element_type=jnp.float32)
        u = jnp.dot(rows, w3_ref[0], preferred_element_type=jnp.float32)
        h = (g * jax.nn.sigmoid(g)) * u
        contrib = jnp.dot(h.astype(jnp.bfloat16), w2_ref[0],
                          preferred_element_type=jnp.float32)
        sl = pl.ds(pl.multiple_of(k * T, T), T)

        @pl.when(f == 0)
        def _init():
            acc_ref[sl, :] = contrib

        @pl.when(f > 0)
        def _accum():
            acc_ref[sl, :] = acc_ref[sl, :] + contrib

        return carry

    lax.fori_loop(0, nt, tile_body, 0)

    @pl.when(f == NF - 1)
    def _flush():
        def flush_tile(k, carry):
            cp = pltpu.make_async_copy(
                acc_ref.at[pl.ds(pl.multiple_of(k * T, T), T), :],
                out_hbm.at[pl.ds(pl.multiple_of(off + k * T, T), T), :],
                st_sem)
            cp.start()
            cp.wait()
            return carry
        lax.fori_loop(0, nt, flush_tile, 0)


def _grouped_ffn(x_sorted, offs, ntiles, w1, w3, w2):
    return pl.pallas_call(
        _grouped_ffn_body,
        grid=(E, NF),
        in_specs=[
            pl.BlockSpec(memory_space=pltpu.SMEM),
            pl.BlockSpec(memory_space=pltpu.SMEM),
            pl.BlockSpec((1, D, FB), lambda e, f: (e, 0, f)),
            pl.BlockSpec((1, D, FB), lambda e, f: (e, 0, f)),
            pl.BlockSpec((1, FB, D), lambda e, f: (e, f, 0)),
            pl.BlockSpec(memory_space=pl.ANY),
        ],
        out_specs=pl.BlockSpec(memory_space=pl.ANY),
        out_shape=jax.ShapeDtypeStruct((P, D), jnp.float32),
        scratch_shapes=[
            pltpu.VMEM((N_TOK, D), jnp.bfloat16),
            pltpu.VMEM((N_TOK, D), jnp.float32),
            pltpu.SemaphoreType.DMA,
            pltpu.SemaphoreType.DMA,
        ],
    )(offs, ntiles, w1, w3, w2, x_sorted)


def kernel(x, expert_idx, w1, w3, w2):
    b, s, d = x.shape
    x_flat = x.reshape(-1, d)
    idx = expert_idx.reshape(-1).astype(jnp.int32)
    n = idx.shape[0]

    order = jnp.argsort(idx)
    sorted_e = jnp.take(idx, order)
    counts = jnp.sum(jax.nn.one_hot(idx, E, dtype=jnp.int32), axis=0)
    padded = ((counts + T - 1) // T) * T
    offs = jnp.concatenate([jnp.zeros((1,), jnp.int32),
                            jnp.cumsum(padded)[:-1].astype(jnp.int32)])
    ntiles = (padded // T).astype(jnp.int32)
    cum = jnp.concatenate([jnp.zeros((1,), jnp.int32),
                           jnp.cumsum(counts)[:-1].astype(jnp.int32)])
    dst = jnp.take(offs, sorted_e) + (jnp.arange(n, dtype=jnp.int32)
                                      - jnp.take(cum, sorted_e))
    src_of_pos = jnp.zeros((P,), jnp.int32).at[dst].set(order.astype(jnp.int32))
    pos_of_token = jnp.zeros((n,), jnp.int32).at[order].set(dst)

    x_bf = x_flat.astype(jnp.bfloat16)
    x_sorted = jnp.take(x_bf, src_of_pos, axis=0)

    y_sorted = _grouped_ffn(x_sorted,
                            offs, ntiles,
                            w1.astype(jnp.bfloat16),
                            w3.astype(jnp.bfloat16),
                            w2.astype(jnp.bfloat16))

    out = jnp.take(y_sorted, pos_of_token, axis=0)
    return out.reshape(b, s, d)
